# baseline (speedup 1.0000x reference)
import numpy as np
import jax
import jax.numpy as jnp
from jax import lax
from jax.experimental import pallas as pl
from jax.experimental.pallas import tpu as pltpu

VOCAB = 32000
EMB = 256
MAXSENT = 256
MAXMEM = 256
SEQ = 512
HID = 256
NBLOCKS = 32
PAD = 0

SCORE_CHUNK = 4096
VCHUNK = 3200


def _pos_encoding():
    pos = np.arange(SEQ)[:, None].astype(np.float64)
    i = np.arange(EMB)[None, :]
    angle = pos / np.power(10000.0, 2.0 * (i // 2) / EMB)
    pe = np.zeros((SEQ, EMB))
    pe[:, 0::2] = np.sin(angle[:, 0::2])
    pe[:, 1::2] = np.cos(angle[:, 1::2])
    return jnp.asarray(pe, dtype=jnp.float32)


def _score_body(w1_ref, ft_ref, sb1_ref, w2_ref, sb2_ref, s_out, acc_ref):
    c = pl.program_id(0)

    @pl.when(c == 0)
    def _():
        acc_ref[...] = jnp.zeros_like(acc_ref)

    acc_ref[...] += jnp.dot(w1_ref[...], ft_ref[...],
                            preferred_element_type=jnp.float32)

    @pl.when(c == pl.num_programs(0) - 1)
    def _():
        h = jnp.maximum(acc_ref[...] + sb1_ref[...], 0.0)
        s_out[...] = jax.nn.sigmoid(
            jnp.dot(w2_ref[...], h, preferred_element_type=jnp.float32)
            + sb2_ref[...])


def _score_net(sW1, flat_t, sb1_col, sW2, sb2_col):
    nsteps = sW1.shape[1] // SCORE_CHUNK
    return pl.pallas_call(
        _score_body,
        grid=(nsteps,),
        in_specs=[
            pl.BlockSpec((64, SCORE_CHUNK), lambda c: (0, c)),
            pl.BlockSpec((SCORE_CHUNK, 2), lambda c: (c, 0)),
            pl.BlockSpec((64, 1), lambda c: (0, 0)),
            pl.BlockSpec((MAXSENT, 64), lambda c: (0, 0)),
            pl.BlockSpec((MAXSENT, 1), lambda c: (0, 0)),
        ],
        out_specs=pl.BlockSpec((MAXSENT, 2), lambda c: (0, 0)),
        out_shape=jax.ShapeDtypeStruct((MAXSENT, 2), jnp.float32),
        scratch_shapes=[pltpu.VMEM((64, 2), jnp.float32)],
    )(sW1, flat_t, sb1_col, sW2, sb2_col)


def _rank_body(sc_ref, sr_ref, tokc_ref, tokr_ref, toks_out, scores_out):
    i0 = lax.broadcasted_iota(jnp.int32, (SEQ, SEQ), 0)
    i1 = lax.broadcasted_iota(jnp.int32, (SEQ, SEQ), 1)
    tok_c = tokc_ref[...]
    tok_r = tokr_ref[...]
    tokeq = tok_c == tok_r
    dup_c = jnp.sum(jnp.where(tokeq & (i1 > i0), 1.0, 0.0),
                    axis=1, keepdims=True)
    dup_r = jnp.sum(jnp.where(tokeq & (i0 > i1), 1.0, 0.0),
                    axis=0, keepdims=True)
    live_c = (dup_c == 0.0) & (tok_c != PAD)
    live_r = (dup_r == 0.0) & (tok_r != PAD)
    neg = jnp.float32(-1e20)
    s2c = jnp.where(live_c, sc_ref[...], neg)
    s2r = jnp.where(live_r, sr_ref[...], neg)
    t2c = jnp.where(live_c, tok_c, 0)
    t2r = jnp.where(live_r, tok_r, 0)

    prec2 = (s2r > s2c) | ((s2r == s2c) & ((t2r < t2c) |
                                           ((t2r == t2c) & (i1 < i0))))
    rank_col = jnp.sum(jnp.where(prec2, 1.0, 0.0),
                       axis=1, keepdims=True)
    onehot = jnp.where(rank_col == i1.astype(jnp.float32), 1.0, 0.0)
    outs = jnp.dot(s2r, onehot, preferred_element_type=jnp.float32,
                   precision=lax.Precision.HIGHEST)
    outt = jnp.dot(t2r.astype(jnp.float32), onehot,
                   preferred_element_type=jnp.float32,
                   precision=lax.Precision.HIGHEST)
    scores_out[...] = outs[:, :MAXMEM]
    toks_out[...] = outt[:, :MAXMEM].astype(jnp.int32)


def _rank_scatter(s_col, s_row, tok_col, tok_row):
    return pl.pallas_call(
        _rank_body,
        in_specs=[
            pl.BlockSpec((SEQ, 1), lambda: (0, 0)),
            pl.BlockSpec((1, SEQ), lambda: (0, 0)),
            pl.BlockSpec((SEQ, 1), lambda: (0, 0)),
            pl.BlockSpec((1, SEQ), lambda: (0, 0)),
        ],
        out_specs=[
            pl.BlockSpec((1, MAXMEM), lambda: (0, 0)),
            pl.BlockSpec((1, MAXMEM), lambda: (0, 0)),
        ],
        out_shape=[
            jax.ShapeDtypeStruct((1, MAXMEM), jnp.int32),
            jax.ShapeDtypeStruct((1, MAXMEM), jnp.float32),
        ],
    )(s_col, s_row, tok_col, tok_row)


def _dec_body(xl_ref, lng_ref, lnb_ref, w1_ref, b1_ref, w2_ref, b2_ref,
              w3_ref, b3_ref, out_ref, h2s):
    i = pl.program_id(0)

    @pl.when(i == 0)
    def _():
        x = xl_ref[...]
        mu = jnp.mean(x)
        var = jnp.mean((x - mu) ** 2)
        ln = (x - mu) / jnp.sqrt(var + 1e-5) * lng_ref[...] + lnb_ref[...]
        h1 = jnp.maximum(
            jnp.dot(ln, w1_ref[...], preferred_element_type=jnp.float32)
            + b1_ref[...], 0.0)
        h2 = jnp.maximum(
            jnp.dot(h1, w2_ref[...], preferred_element_type=jnp.float32)
            + b2_ref[...], 0.0)
        h2s[...] = h2

    out_ref[...] = jnp.dot(h2s[...], w3_ref[...],
                           preferred_element_type=jnp.float32) + b3_ref[...]


def _decoder(xl, lng, lnb, dW1_t, db1, dW2_t, db2, dW3_t, db3):
    return pl.pallas_call(
        _dec_body,
        grid=(VOCAB // VCHUNK,),
        in_specs=[
            pl.BlockSpec((1, HID), lambda i: (0, 0)),
            pl.BlockSpec((1, HID), lambda i: (0, 0)),
            pl.BlockSpec((1, HID), lambda i: (0, 0)),
            pl.BlockSpec((HID, HID), lambda i: (0, 0)),
            pl.BlockSpec((1, HID), lambda i: (0, 0)),
            pl.BlockSpec((HID, HID), lambda i: (0, 0)),
            pl.BlockSpec((1, HID), lambda i: (0, 0)),
            pl.BlockSpec((HID, VCHUNK), lambda i: (0, i)),
            pl.BlockSpec((1, VCHUNK), lambda i: (0, i)),
        ],
        out_specs=pl.BlockSpec((1, VCHUNK), lambda i: (0, i)),
        out_shape=jax.ShapeDtypeStruct((1, VOCAB), jnp.float32),
        scratch_shapes=[pltpu.VMEM((1, HID), jnp.float32)],
    )(xl, lng, lnb, dW1_t, db1, dW2_t, db2, dW3_t, db3)


def kernel(input_tokens, memory_context, emb_table, sW1, sb1, sW2, sb2,
           Wih, Whh, bih, bhh, Wlin, blin, enc_lng, enc_lnb, dec_lng,
           dec_lnb, dW1, db1, dW2, db2, dW3, db3):
    padded = jnp.pad(input_tokens, (0, MAXSENT - input_tokens.shape[0]),
                     constant_values=PAD)
    combined = jnp.concatenate([padded, memory_context])

    gemb = emb_table[combined]
    flat_t = jnp.stack([gemb[:MAXSENT].reshape(-1),
                        gemb[MAXSENT:].reshape(-1)], axis=1)

    s2d = _score_net(sW1, flat_t, sb1.reshape(64, 1), sW2,
                     sb2.reshape(MAXSENT, 1))
    s512 = s2d.T.reshape(SEQ)
    toks2d, scores2d = _rank_scatter(
        s512.reshape(SEQ, 1), s512.reshape(1, SEQ),
        combined.reshape(SEQ, 1), combined.reshape(1, SEQ))
    mem_toks = toks2d.reshape(MAXMEM)
    mem_scores = scores2d.reshape(MAXMEM)

    seq = jnp.concatenate([mem_toks, padded])
    x0 = emb_table[seq]

    def _gru_layer_j(xseq, h0, Wih_l, Whh_l, bih_l, bhh_l):
        def step(h, x):
            gi = Wih_l @ x + bih_l
            gh = Whh_l @ h + bhh_l
            ir, iz, inn = jnp.split(gi, 3)
            hr, hz, hn = jnp.split(gh, 3)
            r = jax.nn.sigmoid(ir + hr)
            z = jax.nn.sigmoid(iz + hz)
            n = jnp.tanh(inn + r * hn)
            hnew = (1.0 - z) * n + z * h
            return hnew, hnew
        hT, ys = lax.scan(step, h0, xseq)
        return ys, hT

    def _enc_j(x):
        h0 = jnp.zeros((2, HID), dtype=x.dtype)
        def block(carry, params):
            x, h = carry
            wih, whh, bi, bh, wl, bl, g, b = params
            y, ha = _gru_layer_j(x, h[0], wih[0], whh[0], bi[0], bh[0])
            y, hb = _gru_layer_j(y, h[1], wih[1], whh[1], bi[1], bh[1])
            y = y @ wl.T + bl
            mu = y.mean(); var = y.var()
            y = (y - mu) / jnp.sqrt(var + 1e-5) * g + b
            return (y, jnp.stack([ha, hb])), None
        (x, _), _ = lax.scan(block, (x, h0),
                             (Wih, Whh, bih, bhh, Wlin, blin, enc_lng, enc_lnb))
        return x
    xl = _enc_j(x0 + _pos_encoding())[SEQ - 1:SEQ, :]

    logits2d = _decoder(
        xl, dec_lng.reshape(1, HID), dec_lnb.reshape(1, HID),
        dW1.T, db1.reshape(1, HID), dW2.T, db2.reshape(1, HID),
        dW3.T, db3.reshape(1, VOCAB))
    return logits2d.reshape(VOCAB), mem_toks, mem_scores
